# baseline (speedup 1.0000x reference)
import functools

import jax
import jax.numpy as jnp
from jax import lax
from jax.experimental import pallas as pl
from jax.experimental.pallas import tpu as pltpu
from jax.experimental.pallas import tpu_sc as plsc

HIDDEN = 1024
INTER = 4096
K = 8
L = 16
P = 1024
S = 1024
NPASS = 4
NBUCKET = 2 ** K
EPS = 1e-12
MARGIN = 1.0
TOP = 32

_F32 = jnp.float32
_HI = lax.Precision.HIGHEST


def _code_matrix():
    cidx = lax.broadcasted_iota(jnp.int32, (K * L, L), 0)
    lidx = lax.broadcasted_iota(jnp.int32, (K * L, L), 1)
    pw = jnp.left_shift(jnp.int32(1), cidx % K).astype(_F32)
    return jnp.where(cidx // K == lidx, pw, 0.0)


def _prep_body(hid_ref, mu_ref, var_ref, gamma_ref, beta_ref, hw_ref, w1_ref,
               normed_ref, cq_ref, cw_ref):
    x = hid_ref[...]
    normed = ((x - mu_ref[...]) / jnp.sqrt(var_ref[...] + EPS)
              * gamma_ref[...] + beta_ref[...])
    normed_ref[...] = normed
    m = _code_matrix()
    hw = hw_ref[...]
    proj_q = jnp.dot(normed, hw)
    cq_ref[...] = jnp.dot((proj_q > 0).astype(_F32), m)
    proj_w = jnp.dot(w1_ref[...], hw)
    cw_ref[...] = jnp.dot((proj_w > 0).astype(_F32), m)


def _cumsum_32x128(x, t128, t32s):
    row = jnp.dot(x, t128, precision=_HI)
    offs = jnp.dot(t32s, jnp.sum(x, axis=1, keepdims=True), precision=_HI)
    return row + offs


def _scores_body(cq_ref, cw_ref, scores_ref):
    cq = cq_ref[...]
    cw = cw_ref[...]
    iota_b = lax.broadcasted_iota(jnp.int32, (1, NBUCKET), 1).astype(_F32)
    scores4 = jnp.zeros((NPASS, INTER), _F32)
    for l in range(L):
        oq = (cq[:, l:l + 1] == iota_b).astype(_F32)
        cnt = jnp.concatenate(
            [jnp.sum(oq[p * P:(p + 1) * P], axis=0, keepdims=True)
             for p in range(NPASS)], axis=0)
        sel = (cw[:, l:l + 1] == iota_b).astype(_F32)
        cnt_hi = jnp.floor(cnt * 0.25)
        cnt_lo = cnt - 4.0 * cnt_hi
        dn = (((1,), (1,)), ((), ()))
        scores4 = (scores4 + 4.0 * lax.dot_general(cnt_hi, sel, dn)
                   + lax.dot_general(cnt_lo, sel, dn))
    scores_ref[...] = scores4.reshape(NPASS, 1, INTER)


def _pick_body(scores_ref, ids_ref):
    sc2 = scores_ref[...].reshape(32, 128)

    def bs_body(_, c):
        lo, hi = c
        mid = lo + (hi - lo) // 2
        cge = jnp.sum((sc2 >= mid.astype(_F32)).astype(_F32))
        take = cge >= float(S)
        return (jnp.where(take, mid, lo), jnp.where(take, hi, mid))

    lo, _ = lax.fori_loop(0, 17, bs_body,
                          (jnp.int32(0), jnp.int32(L * P + 1)))
    tauf = lo.astype(_F32)

    ii = lax.broadcasted_iota(jnp.int32, (128, 128), 0)
    jj = lax.broadcasted_iota(jnp.int32, (128, 128), 1)
    t128 = (ii <= jj).astype(_F32)
    t32s = (jj[:32, :32] < ii[:32, :32]).astype(_F32)

    m_gt = (sc2 > tauf).astype(_F32)
    n_gt = jnp.sum(m_gt)
    tie = (sc2 == tauf).astype(_F32)
    tie_excl = _cumsum_32x128(tie, t128, t32s) - tie
    m = m_gt + tie * (tie_excl < (float(S) - n_gt)).astype(_F32)
    pos_excl = _cumsum_32x128(m, t128, t32s) - m

    pose_flat = pos_excl.reshape(1, INTER)
    m_flat = m.reshape(1, INTER)
    iota_s = lax.broadcasted_iota(jnp.int32, (S, 1), 0).astype(_F32)
    e = (iota_s == pose_flat).astype(_F32) * m_flat
    jall = lax.broadcasted_iota(jnp.int32, (INTER, 1), 0)
    jhi = (jall // 64).astype(_F32)
    jlo = (jall % 64).astype(_F32)
    ids = jnp.dot(e, jhi) * 64.0 + jnp.dot(e, jlo)
    ids_ref[...] = ids.astype(jnp.int32).reshape(1, 1, S)


def _ordered_key(x):
    u = lax.bitcast_convert_type(x, jnp.int32)
    return u ^ ((u >> 31) & jnp.int32(0x7FFFFFFF))


def _key_to_float(k):
    return lax.bitcast_convert_type(k ^ ((k >> 31) & jnp.int32(0x7FFFFFFF)),
                                    _F32)


def _erf(x):
    return lax.erf(x)


def _ffn_body(h_ref, sw1_ref, sb1_ref, sw2_ref, b2_ref, out_ref, loss_ref):
    h = h_ref[...]
    logits = lax.dot_general(h, sw1_ref[...],
                             (((1,), (1,)), ((), ()))) + sb1_ref[0]
    keys = _ordered_key(logits)

    int_min = jnp.int32(-2 ** 31)
    int_max = jnp.int32(2 ** 31 - 1)
    ones = jnp.ones((P, 1), jnp.int32)

    def bs_body(_, c):
        lo1, hi1, lo2, hi2 = c
        mid1 = (lo1 >> 1) + (hi1 >> 1) + (lo1 & hi1 & 1)
        cge = jnp.sum((keys >= mid1).astype(_F32), axis=1, keepdims=True)
        p1 = cge >= float(TOP)
        lo1 = jnp.where(p1, mid1, lo1)
        hi1 = jnp.where(p1, hi1, mid1)
        mid2 = (lo2 >> 1) + (hi2 >> 1) + (lo2 & hi2 & 1)
        cle = jnp.sum((keys <= mid2).astype(_F32), axis=1, keepdims=True)
        p2 = cle >= float(TOP)
        hi2 = jnp.where(p2, mid2, hi2)
        lo2 = jnp.where(p2, lo2, mid2)
        return lo1, hi1, lo2, hi2

    lo1, _, _, hi2 = lax.fori_loop(
        0, 33, bs_body,
        (ones * int_min, ones * int_max, ones * int_min, ones * int_max))
    tau, sig = lo1, hi2
    tauf, sigf = _key_to_float(tau), _key_to_float(sig)
    gt = (keys > tau).astype(_F32)
    top_sum = (jnp.sum(gt * logits, axis=1, keepdims=True)
               + (float(TOP) - jnp.sum(gt, axis=1, keepdims=True)) * tauf)
    lt = (keys < sig).astype(_F32)
    bot_sum = (jnp.sum(lt * logits, axis=1, keepdims=True)
               + (float(TOP) - jnp.sum(lt, axis=1, keepdims=True)) * sigf)
    hinge = jnp.maximum(0.0, MARGIN - (top_sum - bot_sum) / float(TOP))
    loss_ref[...] = jnp.broadcast_to(jnp.sum(hinge) / float(P), (1, 1, 128))

    act = 0.5 * logits * (1.0 + _erf(logits * (2.0 ** -0.5)))
    out_ref[...] = jnp.dot(act, sw2_ref[...]) + b2_ref[...]


def _prep(x, mu, var, gamma, beta, hw, w1):
    t = x.shape[0]
    return pl.pallas_call(
        _prep_body,
        grid=(NPASS,),
        in_specs=[
            pl.BlockSpec((P, HIDDEN), lambda p: (p, 0)),
            pl.BlockSpec((P, 1), lambda p: (p, 0)),
            pl.BlockSpec((P, 1), lambda p: (p, 0)),
            pl.BlockSpec((1, HIDDEN), lambda p: (0, 0)),
            pl.BlockSpec((1, HIDDEN), lambda p: (0, 0)),
            pl.BlockSpec((HIDDEN, K * L), lambda p: (0, 0)),
            pl.BlockSpec((INTER // NPASS, HIDDEN), lambda p: (p, 0)),
        ],
        out_specs=[
            pl.BlockSpec((P, HIDDEN), lambda p: (p, 0)),
            pl.BlockSpec((P, L), lambda p: (p, 0)),
            pl.BlockSpec((INTER // NPASS, L), lambda p: (p, 0)),
        ],
        out_shape=[
            jax.ShapeDtypeStruct((t, HIDDEN), _F32),
            jax.ShapeDtypeStruct((t, L), _F32),
            jax.ShapeDtypeStruct((INTER, L), _F32),
        ],
    )(x, mu, var, gamma, beta, hw, w1)


def _select(cq, cw):
    scores4 = pl.pallas_call(
        _scores_body,
        out_shape=jax.ShapeDtypeStruct((NPASS, 1, INTER), _F32),
    )(cq, cw)
    return pl.pallas_call(
        _pick_body,
        grid=(NPASS,),
        in_specs=[pl.BlockSpec((1, 1, INTER), lambda p: (p, 0, 0))],
        out_specs=pl.BlockSpec((1, 1, S), lambda p: (p, 0, 0)),
        out_shape=jax.ShapeDtypeStruct((NPASS, 1, S), jnp.int32),
    )(scores4)


def _sc_gather_p(w1, b1, w2, ids_p):
    info = plsc.get_sparse_core_info()
    nc, ns = info.num_cores, info.num_subcores
    nw = nc * ns
    per = S // nw
    mesh = plsc.VectorSubcoreMesh(core_axis_name="c", subcore_axis_name="s")

    @functools.partial(
        pl.kernel,
        mesh=mesh,
        out_type=[
            jax.ShapeDtypeStruct((S, HIDDEN), _F32),
            jax.ShapeDtypeStruct((S, HIDDEN), _F32),
            jax.ShapeDtypeStruct((S,), _F32),
        ],
        scratch_types=[
            pltpu.VMEM((per,), jnp.int32),
            pltpu.VMEM((per, HIDDEN), _F32),
            pltpu.VMEM((per, HIDDEN), _F32),
            pltpu.VMEM((INTER,), _F32),
            pltpu.VMEM((per,), _F32),
            pltpu.SemaphoreType.DMA,
            pltpu.SemaphoreType.DMA,
        ],
        compiler_params=pltpu.CompilerParams(needs_layout_passes=False),
    )
    def k(w1_hbm, b1_hbm, w2_hbm, ids_hbm, sw1_hbm, sw2_hbm, sb1_hbm,
          idx_v, rows1_v, rows2_v, b1_v, sb_v, sem1, sem2):
        wid = lax.axis_index("s") * nc + lax.axis_index("c")
        base = wid * per
        pltpu.sync_copy(ids_hbm.at[pl.ds(base, per)], idx_v)
        cp1 = pltpu.async_copy(w1_hbm.at[idx_v], rows1_v, sem1)
        cp2 = pltpu.async_copy(w2_hbm.at[idx_v], rows2_v, sem2)
        pltpu.sync_copy(b1_hbm, b1_v)
        for g in range(per // 16):
            idx16 = idx_v[pl.ds(g * 16, 16)]
            sb_v[pl.ds(g * 16, 16)] = plsc.load_gather(b1_v, [idx16])
        pltpu.sync_copy(sb_v, sb1_hbm.at[pl.ds(base, per)])
        cp1.wait()
        pltpu.sync_copy(rows1_v, sw1_hbm.at[pl.ds(base, per)])
        cp2.wait()
        pltpu.sync_copy(rows2_v, sw2_hbm.at[pl.ds(base, per)])

    return k(w1, b1, w2, ids_p)


def _ffn_p(h, sw1, sb1, sw2, b2):
    return pl.pallas_call(
        _ffn_body,
        out_shape=[
            jax.ShapeDtypeStruct((P, HIDDEN), _F32),
            jax.ShapeDtypeStruct((1, 1, 128), _F32),
        ],
    )(h, sw1, sb1, sw2, b2)


def kernel(hidden_states, ln_gamma, ln_beta, W1, b1, hash_weight, W2, b2):
    shape = hidden_states.shape
    x = hidden_states.reshape(-1, HIDDEN)
    mu = x.mean(-1, keepdims=True)
    var = ((x - mu) ** 2).mean(-1, keepdims=True)
    normed, cq, cw = _prep(x, mu, var, ln_gamma.reshape(1, -1),
                           ln_beta.reshape(1, -1), hash_weight, W1)
    ids = _select(cq, cw).reshape(NPASS, S)
    b2r = b2.reshape(1, -1)
    outs, losses = [], []
    for p in range(NPASS):
        sw1, sw2, sb1 = _sc_gather_p(W1, b1, W2, ids[p])
        o, l = _ffn_p(normed[p * P:(p + 1) * P], sw1, sb1.reshape(1, 1, S),
                      sw2, b2r)
        outs.append(o)
        losses.append(l[0, 0, 0])
    out = jnp.concatenate(outs, axis=0)
    return out.reshape(shape), jnp.mean(jnp.stack(losses))


# speedup vs baseline: 1.0468x; 1.0468x over previous
import functools

import jax
import jax.numpy as jnp
from jax import lax
from jax.experimental import pallas as pl
from jax.experimental.pallas import tpu as pltpu
from jax.experimental.pallas import tpu_sc as plsc

HIDDEN = 1024
INTER = 4096
K = 8
L = 16
P = 1024
S = 1024
NPASS = 4
NBUCKET = 2 ** K
EPS = 1e-12
MARGIN = 1.0
TOP = 32

_F32 = jnp.float32
_HI = lax.Precision.HIGHEST


def _code_matrix():
    cidx = lax.broadcasted_iota(jnp.int32, (K * L, L), 0)
    lidx = lax.broadcasted_iota(jnp.int32, (K * L, L), 1)
    pw = jnp.left_shift(jnp.int32(1), cidx % K).astype(_F32)
    return jnp.where(cidx // K == lidx, pw, 0.0)


def _prep_body(hid_ref, mu_ref, var_ref, gamma_ref, beta_ref, hw_ref, w1_ref,
               normed_ref, cq_ref, cw_ref):
    x = hid_ref[...]
    normed = ((x - mu_ref[...]) / jnp.sqrt(var_ref[...] + EPS)
              * gamma_ref[...] + beta_ref[...])
    normed_ref[...] = normed
    m = _code_matrix()
    hw = hw_ref[...]
    proj_q = jnp.dot(normed, hw)
    cq_ref[...] = jnp.dot((proj_q > 0).astype(_F32), m)
    proj_w = jnp.dot(w1_ref[...], hw)
    cw_ref[...] = jnp.dot((proj_w > 0).astype(_F32), m)


def _cumsum_32x128(x, t128, t32s):
    row = jnp.dot(x, t128, precision=_HI)
    offs = jnp.dot(t32s, jnp.sum(x, axis=1, keepdims=True), precision=_HI)
    return row + offs


def _scores_body(cq_ref, cw_ref, scores_ref):
    cq = cq_ref[...]
    cw = cw_ref[...]
    iota_b = lax.broadcasted_iota(jnp.int32, (1, NBUCKET), 1).astype(_F32)
    scores4 = jnp.zeros((NPASS, INTER), _F32)
    for l in range(L):
        oq = (cq[:, l:l + 1] == iota_b).astype(_F32)
        cnt = jnp.concatenate(
            [jnp.sum(oq[p * P:(p + 1) * P], axis=0, keepdims=True)
             for p in range(NPASS)], axis=0)
        sel = (cw[:, l:l + 1] == iota_b).astype(_F32)
        cnt_hi = jnp.floor(cnt * 0.25)
        cnt_lo = cnt - 4.0 * cnt_hi
        dn = (((1,), (1,)), ((), ()))
        scores4 = (scores4 + 4.0 * lax.dot_general(cnt_hi, sel, dn)
                   + lax.dot_general(cnt_lo, sel, dn))
    scores_ref[...] = scores4.reshape(NPASS, 1, INTER)


def _pick_body(scores_ref, ids_ref):
    sc2 = scores_ref[...].reshape(32, 128)

    def bs_body(_, c):
        lo, hi = c
        mid = lo + (hi - lo) // 2
        cge = jnp.sum((sc2 >= mid.astype(_F32)).astype(_F32))
        take = cge >= float(S)
        return (jnp.where(take, mid, lo), jnp.where(take, hi, mid))

    lo, _ = lax.fori_loop(0, 17, bs_body,
                          (jnp.int32(0), jnp.int32(L * P + 1)))
    tauf = lo.astype(_F32)

    ii = lax.broadcasted_iota(jnp.int32, (128, 128), 0)
    jj = lax.broadcasted_iota(jnp.int32, (128, 128), 1)
    t128 = (ii <= jj).astype(_F32)
    t32s = (jj[:32, :32] < ii[:32, :32]).astype(_F32)

    m_gt = (sc2 > tauf).astype(_F32)
    n_gt = jnp.sum(m_gt)
    tie = (sc2 == tauf).astype(_F32)
    tie_excl = _cumsum_32x128(tie, t128, t32s) - tie
    m = m_gt + tie * (tie_excl < (float(S) - n_gt)).astype(_F32)
    pos_excl = _cumsum_32x128(m, t128, t32s) - m

    pose_flat = pos_excl.reshape(1, INTER)
    m_flat = m.reshape(1, INTER)
    iota_s = lax.broadcasted_iota(jnp.int32, (S, 1), 0).astype(_F32)
    e = (iota_s == pose_flat).astype(_F32) * m_flat
    jall = lax.broadcasted_iota(jnp.int32, (INTER, 1), 0)
    jhi = (jall // 64).astype(_F32)
    jlo = (jall % 64).astype(_F32)
    ids = jnp.dot(e, jhi) * 64.0 + jnp.dot(e, jlo)
    ids_ref[...] = ids.astype(jnp.int32).reshape(1, 1, S)


def _ordered_key(x):
    u = lax.bitcast_convert_type(x, jnp.int32)
    return u ^ ((u >> 31) & jnp.int32(0x7FFFFFFF))


def _key_to_float(k):
    return lax.bitcast_convert_type(k ^ ((k >> 31) & jnp.int32(0x7FFFFFFF)),
                                    _F32)


def _erf(x):
    return lax.erf(x)


def _ffn_body(h_ref, sw1_ref, sb1_ref, sw2_ref, b2_ref, out_ref, loss_ref):
    h = h_ref[...]
    logits = lax.dot_general(h, sw1_ref[...],
                             (((1,), (1,)), ((), ()))) + sb1_ref[0]
    keys = _ordered_key(logits)

    int_min = jnp.int32(-2 ** 31)
    int_max = jnp.int32(2 ** 31 - 1)
    ones = jnp.ones((P, 1), jnp.int32)

    def bs_body(_, c):
        lo1, hi1, lo2, hi2 = c
        mid1 = (lo1 >> 1) + (hi1 >> 1) + (lo1 & hi1 & 1)
        cge = jnp.sum((keys >= mid1).astype(_F32), axis=1, keepdims=True)
        p1 = cge >= float(TOP)
        lo1 = jnp.where(p1, mid1, lo1)
        hi1 = jnp.where(p1, hi1, mid1)
        mid2 = (lo2 >> 1) + (hi2 >> 1) + (lo2 & hi2 & 1)
        cle = jnp.sum((keys <= mid2).astype(_F32), axis=1, keepdims=True)
        p2 = cle >= float(TOP)
        hi2 = jnp.where(p2, mid2, hi2)
        lo2 = jnp.where(p2, lo2, mid2)
        return lo1, hi1, lo2, hi2

    lo1, _, _, hi2 = lax.fori_loop(
        0, 33, bs_body,
        (ones * int_min, ones * int_max, ones * int_min, ones * int_max))
    tau, sig = lo1, hi2
    tauf, sigf = _key_to_float(tau), _key_to_float(sig)
    gt = (keys > tau).astype(_F32)
    top_sum = (jnp.sum(gt * logits, axis=1, keepdims=True)
               + (float(TOP) - jnp.sum(gt, axis=1, keepdims=True)) * tauf)
    lt = (keys < sig).astype(_F32)
    bot_sum = (jnp.sum(lt * logits, axis=1, keepdims=True)
               + (float(TOP) - jnp.sum(lt, axis=1, keepdims=True)) * sigf)
    hinge = jnp.maximum(0.0, MARGIN - (top_sum - bot_sum) / float(TOP))
    loss_ref[...] = jnp.broadcast_to(jnp.sum(hinge) / float(P), (1, 1, 128))

    act = 0.5 * logits * (1.0 + _erf(logits * (2.0 ** -0.5)))
    out_ref[...] = jnp.dot(act, sw2_ref[...]) + b2_ref[...]


def _prep(x, mu, var, gamma, beta, hw, w1):
    t = x.shape[0]
    return pl.pallas_call(
        _prep_body,
        grid=(NPASS,),
        in_specs=[
            pl.BlockSpec((P, HIDDEN), lambda p: (p, 0)),
            pl.BlockSpec((P, 1), lambda p: (p, 0)),
            pl.BlockSpec((P, 1), lambda p: (p, 0)),
            pl.BlockSpec((1, HIDDEN), lambda p: (0, 0)),
            pl.BlockSpec((1, HIDDEN), lambda p: (0, 0)),
            pl.BlockSpec((HIDDEN, K * L), lambda p: (0, 0)),
            pl.BlockSpec((INTER // NPASS, HIDDEN), lambda p: (p, 0)),
        ],
        out_specs=[
            pl.BlockSpec((P, HIDDEN), lambda p: (p, 0)),
            pl.BlockSpec((P, L), lambda p: (p, 0)),
            pl.BlockSpec((INTER // NPASS, L), lambda p: (p, 0)),
        ],
        out_shape=[
            jax.ShapeDtypeStruct((t, HIDDEN), _F32),
            jax.ShapeDtypeStruct((t, L), _F32),
            jax.ShapeDtypeStruct((INTER, L), _F32),
        ],
    )(x, mu, var, gamma, beta, hw, w1)


def _select(cq, cw):
    scores4 = pl.pallas_call(
        _scores_body,
        out_shape=jax.ShapeDtypeStruct((NPASS, 1, INTER), _F32),
    )(cq, cw)
    return pl.pallas_call(
        _pick_body,
        grid=(NPASS,),
        in_specs=[pl.BlockSpec((1, 1, INTER), lambda p: (p, 0, 0))],
        out_specs=pl.BlockSpec((1, 1, S), lambda p: (p, 0, 0)),
        out_shape=jax.ShapeDtypeStruct((NPASS, 1, S), jnp.int32),
    )(scores4)


def _sc_gather(w1, b1, w2, ids):
    info = plsc.get_sparse_core_info()
    nc, ns = info.num_cores, info.num_subcores
    nw = nc * ns
    per = S // nw
    mesh = plsc.VectorSubcoreMesh(core_axis_name="c", subcore_axis_name="s")

    @functools.partial(
        pl.kernel,
        mesh=mesh,
        out_type=[
            jax.ShapeDtypeStruct((NPASS * S, HIDDEN), _F32),
            jax.ShapeDtypeStruct((NPASS * S, HIDDEN), _F32),
            jax.ShapeDtypeStruct((NPASS, S), _F32),
        ],
        scratch_types=[
            pltpu.VMEM((per,), jnp.int32),
            pltpu.VMEM((per, HIDDEN), _F32),
            pltpu.VMEM((per, HIDDEN), _F32),
            pltpu.VMEM((INTER,), _F32),
            pltpu.VMEM((per,), _F32),
            pltpu.SemaphoreType.DMA,
            pltpu.SemaphoreType.DMA,
        ],
        compiler_params=pltpu.CompilerParams(needs_layout_passes=False),
    )
    def k(w1_hbm, b1_hbm, w2_hbm, ids_hbm, sw1_hbm, sw2_hbm, sb1_hbm,
          idx_v, rows1_v, rows2_v, b1_v, sb_v, sem1, sem2):
        wid = lax.axis_index("s") * nc + lax.axis_index("c")
        base = wid * per
        pltpu.sync_copy(b1_hbm, b1_v)
        for p in range(NPASS):
            pltpu.sync_copy(ids_hbm.at[p, pl.ds(base, per)], idx_v)
            cp1 = pltpu.async_copy(w1_hbm.at[idx_v], rows1_v, sem1)
            cp2 = pltpu.async_copy(w2_hbm.at[idx_v], rows2_v, sem2)
            for g in range(per // 16):
                idx16 = idx_v[pl.ds(g * 16, 16)]
                sb_v[pl.ds(g * 16, 16)] = plsc.load_gather(b1_v, [idx16])
            pltpu.sync_copy(sb_v, sb1_hbm.at[p, pl.ds(base, per)])
            cp1.wait()
            pltpu.sync_copy(rows1_v, sw1_hbm.at[pl.ds(p * S + base, per)])
            cp2.wait()
            pltpu.sync_copy(rows2_v, sw2_hbm.at[pl.ds(p * S + base, per)])

    return k(w1, b1, w2, ids)


def _ffn(normed, sw1, sb1, sw2, b2):
    t = normed.shape[0]
    return pl.pallas_call(
        _ffn_body,
        grid=(NPASS,),
        in_specs=[
            pl.BlockSpec((P, HIDDEN), lambda p: (p, 0)),
            pl.BlockSpec((S, HIDDEN), lambda p: (p, 0)),
            pl.BlockSpec((1, 1, S), lambda p: (p, 0, 0)),
            pl.BlockSpec((S, HIDDEN), lambda p: (p, 0)),
            pl.BlockSpec((1, HIDDEN), lambda p: (0, 0)),
        ],
        out_specs=[
            pl.BlockSpec((P, HIDDEN), lambda p: (p, 0)),
            pl.BlockSpec((1, 1, 128), lambda p: (p, 0, 0)),
        ],
        out_shape=[
            jax.ShapeDtypeStruct((t, HIDDEN), _F32),
            jax.ShapeDtypeStruct((NPASS, 1, 128), _F32),
        ],
    )(normed, sw1, sb1, sw2, b2)


def kernel(hidden_states, ln_gamma, ln_beta, W1, b1, hash_weight, W2, b2):
    shape = hidden_states.shape
    x = hidden_states.reshape(-1, HIDDEN)
    mu = x.mean(-1, keepdims=True)
    var = ((x - mu) ** 2).mean(-1, keepdims=True)
    normed, cq, cw = _prep(x, mu, var, ln_gamma.reshape(1, -1),
                           ln_beta.reshape(1, -1), hash_weight, W1)
    ids = _select(cq, cw)
    sw1, sw2, sb1 = _sc_gather(W1, b1, W2, ids.reshape(NPASS, S))
    out, loss = _ffn(normed, sw1, sb1.reshape(NPASS, 1, S), sw2,
                     b2.reshape(1, -1))
    return out.reshape(shape), jnp.mean(loss[:, 0, 0])
